# baseline (speedup 1.0000x reference)
import jax
import jax.numpy as jnp
from jax import lax
from jax.experimental import pallas as pl
from jax.experimental.pallas import tpu as pltpu
from jax.experimental.pallas import tpu_sc as plsc

VOCAB = 1000000
EMB = 64
EMBP = 128
B = 4096
L = 200

NC = 2
NS = 16
NW = NC * NS

RPW = B // NW
NCHUNK = RPW
NBUF = 4
FIRE_AHEAD = 2
SPLITS = ((0, 128), (128, 72))


def _gather_kernel(table, enc, out, idx_v,
                   rows0, rows1, rows2, rows3,
                   sg0, sg1, sg2, sg3, so0, so1, so2, so3):
    cid = lax.axis_index("c")
    sid = lax.axis_index("s")
    wid = sid * NC + cid
    row0 = wid * RPW

    rows = (rows0, rows1, rows2, rows3)
    sg = (sg0, sg1, sg2, sg3)
    so = (so0, so1, so2, so3)

    pltpu.sync_copy(enc.at[pl.ds(row0, RPW)], idx_v)

    def fire_gathers(c, b):
        for (off, n) in SPLITS:
            pltpu.async_copy(
                table.at[idx_v.at[c, pl.ds(off, n)]],
                rows[b].at[0, pl.ds(off, n)],
                sg[b],
            )

    def wait_gathers(c, b):
        for (off, n) in SPLITS:
            pltpu.make_async_copy(
                table.at[idx_v.at[c, pl.ds(off, n)]],
                rows[b].at[0, pl.ds(off, n)],
                sg[b],
            ).wait()

    def out_copy(c, b):
        return pltpu.make_async_copy(
            rows[b],
            out.at[pl.ds(row0 + c, 1)],
            so[b],
        )

    for b in range(FIRE_AHEAD):
        fire_gathers(b, b)

    def body(i, carry):
        c0 = i * NBUF
        for b in range(NBUF):
            c = c0 + b
            wait_gathers(c, b)
            out_copy(c, b).start()

            @pl.when(c >= FIRE_AHEAD)
            def _():
                out_copy(c - FIRE_AHEAD, (b + FIRE_AHEAD) % NBUF).wait()

            @pl.when(c + FIRE_AHEAD < NCHUNK)
            def _():
                fire_gathers(c + FIRE_AHEAD, (b + FIRE_AHEAD) % NBUF)

        return carry

    lax.fori_loop(0, NCHUNK // NBUF, body, 0)

    for c in range(NCHUNK - FIRE_AHEAD, NCHUNK):
        out_copy(c, c % NBUF).wait()


def _mask_body(enc_ref, mask_ref):
    mask_ref[...] = (enc_ref[...] != 0).astype(jnp.int32)


@jax.jit
def _run(encoded, embeddings):
    t128 = jnp.pad(embeddings, ((0, 0), (0, EMBP - EMB)))
    gather = pl.kernel(
        _gather_kernel,
        out_type=jax.ShapeDtypeStruct((B, L, EMBP), jnp.float32),
        mesh=plsc.VectorSubcoreMesh(core_axis_name="c", subcore_axis_name="s"),
        compiler_params=pltpu.CompilerParams(use_tc_tiling_on_sc=False),
        scratch_types=(
            [pltpu.VMEM((RPW, L), jnp.int32)]
            + [pltpu.VMEM((1, L, EMBP), jnp.float32)
               for _ in range(NBUF)]
            + [pltpu.SemaphoreType.DMA for _ in range(2 * NBUF)]
        ),
    )
    out128 = gather(t128, encoded)
    emb = out128[:, :, :EMB]
    mask = pl.pallas_call(
        _mask_body,
        out_shape=jax.ShapeDtypeStruct((B, L), jnp.int32),
    )(encoded)
    return emb, mask


def kernel(encoded, embeddings):
    return _run(encoded, embeddings)


# speedup vs baseline: 1.0645x; 1.0645x over previous
import jax
import jax.numpy as jnp
from jax import lax
from jax.experimental import pallas as pl
from jax.experimental.pallas import tpu as pltpu
from jax.experimental.pallas import tpu_sc as plsc

VOCAB = 1000000
EMB = 64
EMBP = 128
B = 4096
L = 200

NC = 2
NS = 16
NW = NC * NS

RPW = B // NW
NCHUNK = RPW
NBUF = 4
FIRE_AHEAD = 2
SPLITS = ((0, 128), (128, 72))


def _gather_kernel(table, enc, out, idx_v,
                   rows0, rows1, rows2, rows3,
                   sg0, sg1, sg2, sg3, so0, so1, so2, so3):
    cid = lax.axis_index("c")
    sid = lax.axis_index("s")
    wid = sid * NC + cid
    row0 = wid * RPW

    rows = (rows0, rows1, rows2, rows3)
    sg = (sg0, sg1, sg2, sg3)
    so = (so0, so1, so2, so3)

    pltpu.sync_copy(enc.at[pl.ds(row0, RPW)], idx_v)

    def fire_gathers(c, b):
        for (off, n) in SPLITS:
            pltpu.async_copy(
                table.at[idx_v.at[c, pl.ds(off, n)]],
                rows[b].at[0, pl.ds(off, n)],
                sg[b],
            )

    def wait_gathers(c, b):
        for (off, n) in SPLITS:
            pltpu.make_async_copy(
                table.at[idx_v.at[c, pl.ds(off, n)]],
                rows[b].at[0, pl.ds(off, n)],
                sg[b],
            ).wait()

    def out_copy(c, b):
        return pltpu.make_async_copy(
            rows[b],
            out.at[pl.ds(row0 + c, 1)],
            so[b],
        )

    for b in range(FIRE_AHEAD):
        fire_gathers(b, b)

    def body(i, carry):
        c0 = i * NBUF
        for b in range(NBUF):
            c = c0 + b
            wait_gathers(c, b)
            out_copy(c, b).start()

            @pl.when(c >= FIRE_AHEAD)
            def _():
                out_copy(c - FIRE_AHEAD, (b + FIRE_AHEAD) % NBUF).wait()

            @pl.when(c + FIRE_AHEAD < NCHUNK)
            def _():
                fire_gathers(c + FIRE_AHEAD, (b + FIRE_AHEAD) % NBUF)

        return carry

    lax.fori_loop(0, NCHUNK // NBUF, body, 0)

    for c in range(NCHUNK - FIRE_AHEAD, NCHUNK):
        out_copy(c, c % NBUF).wait()


def _mask_body(enc_ref, mask_ref):
    mask_ref[...] = (enc_ref[...] != 0).astype(jnp.int32)


TBLK = 2048


def _pad_body(tT_ref, out_ref):
    out_ref[:, :EMB] = jnp.swapaxes(tT_ref[...], 0, 1)


def _make_t128(embeddings):
    grid = (VOCAB + TBLK - 1) // TBLK
    return pl.pallas_call(
        _pad_body,
        grid=(grid,),
        in_specs=[pl.BlockSpec((EMB, TBLK), lambda i: (0, i))],
        out_specs=pl.BlockSpec((TBLK, EMBP), lambda i: (i, 0)),
        out_shape=jax.ShapeDtypeStruct((VOCAB, EMBP), jnp.float32),
    )(embeddings.T)


@jax.jit
def _run(encoded, embeddings):
    t128 = _make_t128(embeddings)
    gather = pl.kernel(
        _gather_kernel,
        out_type=jax.ShapeDtypeStruct((B, L, EMBP), jnp.float32),
        mesh=plsc.VectorSubcoreMesh(core_axis_name="c", subcore_axis_name="s"),
        compiler_params=pltpu.CompilerParams(use_tc_tiling_on_sc=False),
        scratch_types=(
            [pltpu.VMEM((RPW, L), jnp.int32)]
            + [pltpu.VMEM((1, L, EMBP), jnp.float32)
               for _ in range(NBUF)]
            + [pltpu.SemaphoreType.DMA for _ in range(2 * NBUF)]
        ),
    )
    out128 = gather(t128, encoded)
    emb = out128[:, :, :EMB]
    mask = pl.pallas_call(
        _mask_body,
        out_shape=jax.ShapeDtypeStruct((B, L), jnp.int32),
    )(encoded)
    return emb, mask


def kernel(encoded, embeddings):
    return _run(encoded, embeddings)


# speedup vs baseline: 1.3426x; 1.2612x over previous
import jax
import jax.numpy as jnp
from jax import lax
from jax.experimental import pallas as pl
from jax.experimental.pallas import tpu as pltpu
from jax.experimental.pallas import tpu_sc as plsc

VOCAB = 1000000
EMB = 64
EMBP = 128
B = 4096
L = 200

NC = 2
NS = 16
NW = NC * NS

RPW = B // NW
NCHUNK = RPW
NBUF = 4
FIRE_AHEAD = 2
SPLITS = ((0, 128), (128, 72))


def _gather_kernel(table, enc, out, idx_v,
                   rows0, rows1, rows2, rows3,
                   sg0, sg1, sg2, sg3, so0, so1, so2, so3):
    cid = lax.axis_index("c")
    sid = lax.axis_index("s")
    wid = sid * NC + cid
    row0 = wid * RPW

    rows = (rows0, rows1, rows2, rows3)
    sg = (sg0, sg1, sg2, sg3)
    so = (so0, so1, so2, so3)

    pltpu.sync_copy(enc.at[pl.ds(row0, RPW)], idx_v)

    def fire_gathers(c, b):
        for (off, n) in SPLITS:
            pltpu.async_copy(
                table.at[idx_v.at[c, pl.ds(off, n)]],
                rows[b].at[0, pl.ds(off, n)],
                sg[b],
            )

    def wait_gathers(c, b):
        for (off, n) in SPLITS:
            pltpu.make_async_copy(
                table.at[idx_v.at[c, pl.ds(off, n)]],
                rows[b].at[0, pl.ds(off, n)],
                sg[b],
            ).wait()

    def out_copy(c, b):
        return pltpu.make_async_copy(
            rows[b],
            out.at[pl.ds(row0 + c, 1)],
            so[b],
        )

    for b in range(FIRE_AHEAD):
        fire_gathers(b, b)

    def body(i, carry):
        c0 = i * NBUF
        for b in range(NBUF):
            c = c0 + b
            wait_gathers(c, b)
            out_copy(c, b).start()

            @pl.when(c >= FIRE_AHEAD)
            def _():
                out_copy(c - FIRE_AHEAD, (b + FIRE_AHEAD) % NBUF).wait()

            @pl.when(c + FIRE_AHEAD < NCHUNK)
            def _():
                fire_gathers(c + FIRE_AHEAD, (b + FIRE_AHEAD) % NBUF)

        return carry

    lax.fori_loop(0, NCHUNK // NBUF, body, 0)

    for c in range(NCHUNK - FIRE_AHEAD, NCHUNK):
        out_copy(c, c % NBUF).wait()


def _mask_body(enc_ref, mask_ref):
    mask_ref[...] = (enc_ref[...] != 0).astype(jnp.int32)


TBLK = 8192


def _pad_body(tT_ref, out_ref):
    out_ref[:, :EMB] = jnp.swapaxes(tT_ref[...], 0, 1)


def _make_t128(embeddings):
    grid = (VOCAB + TBLK - 1) // TBLK
    return pl.pallas_call(
        _pad_body,
        grid=(grid,),
        in_specs=[pl.BlockSpec((EMB, TBLK), lambda i: (0, i))],
        out_specs=pl.BlockSpec((TBLK, EMBP), lambda i: (i, 0)),
        out_shape=jax.ShapeDtypeStruct((VOCAB, EMBP), jnp.float32),
    )(embeddings.T)


@jax.jit
def _run(encoded, embeddings):
    t128 = _make_t128(embeddings)
    gather = pl.kernel(
        _gather_kernel,
        out_type=jax.ShapeDtypeStruct((B, L, EMBP), jnp.float32),
        mesh=plsc.VectorSubcoreMesh(core_axis_name="c", subcore_axis_name="s"),
        compiler_params=pltpu.CompilerParams(use_tc_tiling_on_sc=False),
        scratch_types=(
            [pltpu.VMEM((RPW, L), jnp.int32)]
            + [pltpu.VMEM((1, L, EMBP), jnp.float32)
               for _ in range(NBUF)]
            + [pltpu.SemaphoreType.DMA for _ in range(2 * NBUF)]
        ),
    )
    out128 = gather(t128, encoded)
    emb = out128[:, :, :EMB]
    mask = pl.pallas_call(
        _mask_body,
        out_shape=jax.ShapeDtypeStruct((B, L), jnp.int32),
    )(encoded)
    return emb, mask


def kernel(encoded, embeddings):
    return _run(encoded, embeddings)


# speedup vs baseline: 1.3761x; 1.0250x over previous
import jax
import jax.numpy as jnp
from jax import lax
from jax.experimental import pallas as pl
from jax.experimental.pallas import tpu as pltpu
from jax.experimental.pallas import tpu_sc as plsc

VOCAB = 1000000
EMB = 64
EMBP = 128
B = 4096
L = 200

NC = 2
NS = 16
NW = NC * NS

RPW = B // NW
NCHUNK = RPW
NBUF = 4
FIRE_AHEAD = 2
SPLITS = ((0, 128), (128, 72))


def _gather_kernel(table, enc, out, idx_v,
                   rows0, rows1, rows2, rows3,
                   sg0, sg1, sg2, sg3, so0, so1, so2, so3):
    cid = lax.axis_index("c")
    sid = lax.axis_index("s")
    wid = sid * NC + cid
    row0 = wid * RPW

    rows = (rows0, rows1, rows2, rows3)
    sg = (sg0, sg1, sg2, sg3)
    so = (so0, so1, so2, so3)

    pltpu.sync_copy(enc.at[pl.ds(row0, RPW)], idx_v)

    def fire_gathers(c, b):
        for (off, n) in SPLITS:
            pltpu.async_copy(
                table.at[idx_v.at[c, pl.ds(off, n)]],
                rows[b].at[0, pl.ds(off, n)],
                sg[b],
            )

    def wait_gathers(c, b):
        for (off, n) in SPLITS:
            pltpu.make_async_copy(
                table.at[idx_v.at[c, pl.ds(off, n)]],
                rows[b].at[0, pl.ds(off, n)],
                sg[b],
            ).wait()

    def out_copy(c, b):
        return pltpu.make_async_copy(
            rows[b],
            out.at[pl.ds(row0 + c, 1)],
            so[b],
        )

    for b in range(FIRE_AHEAD):
        fire_gathers(b, b)

    def body(i, carry):
        c0 = i * NBUF
        for b in range(NBUF):
            c = c0 + b
            wait_gathers(c, b)
            out_copy(c, b).start()

            @pl.when(c >= FIRE_AHEAD)
            def _():
                out_copy(c - FIRE_AHEAD, (b + FIRE_AHEAD) % NBUF).wait()

            @pl.when(c + FIRE_AHEAD < NCHUNK)
            def _():
                fire_gathers(c + FIRE_AHEAD, (b + FIRE_AHEAD) % NBUF)

        return carry

    lax.fori_loop(0, NCHUNK // NBUF, body, 0)

    for c in range(NCHUNK - FIRE_AHEAD, NCHUNK):
        out_copy(c, c % NBUF).wait()


def _mask_body(enc_ref, mask_ref):
    mask_ref[...] = (enc_ref[...] != 0).astype(jnp.int32)


TBLK = 16384


def _pad_body(tT_ref, out_ref):
    out_ref[:, :EMB] = jnp.swapaxes(tT_ref[...], 0, 1)


def _make_t128(embeddings):
    grid = (VOCAB + TBLK - 1) // TBLK
    return pl.pallas_call(
        _pad_body,
        grid=(grid,),
        in_specs=[pl.BlockSpec((EMB, TBLK), lambda i: (0, i))],
        out_specs=pl.BlockSpec((TBLK, EMBP), lambda i: (i, 0)),
        out_shape=jax.ShapeDtypeStruct((VOCAB, EMBP), jnp.float32),
    )(embeddings.T)


@jax.jit
def _run(encoded, embeddings):
    t128 = _make_t128(embeddings)
    gather = pl.kernel(
        _gather_kernel,
        out_type=jax.ShapeDtypeStruct((B, L, EMBP), jnp.float32),
        mesh=plsc.VectorSubcoreMesh(core_axis_name="c", subcore_axis_name="s"),
        compiler_params=pltpu.CompilerParams(use_tc_tiling_on_sc=False),
        scratch_types=(
            [pltpu.VMEM((RPW, L), jnp.int32)]
            + [pltpu.VMEM((1, L, EMBP), jnp.float32)
               for _ in range(NBUF)]
            + [pltpu.SemaphoreType.DMA for _ in range(2 * NBUF)]
        ),
    )
    out128 = gather(t128, encoded)
    emb = out128[:, :, :EMB]
    mask = pl.pallas_call(
        _mask_body,
        out_shape=jax.ShapeDtypeStruct((B, L), jnp.int32),
    )(encoded)
    return emb, mask


def kernel(encoded, embeddings):
    return _run(encoded, embeddings)


# speedup vs baseline: 1.3863x; 1.0074x over previous
import jax
import jax.numpy as jnp
from jax import lax
from jax.experimental import pallas as pl
from jax.experimental.pallas import tpu as pltpu
from jax.experimental.pallas import tpu_sc as plsc

VOCAB = 1000000
EMB = 64
EMBP = 128
B = 4096
L = 200

NC = 2
NS = 16
NW = NC * NS

RPW = B // NW
NCHUNK = RPW
NBUF = 4
FIRE_AHEAD = 2
SPLITS = ((0, 128), (128, 72))


def _gather_kernel(table, enc, out, idx_v,
                   rows0, rows1, rows2, rows3,
                   sg0, sg1, sg2, sg3, so0, so1, so2, so3):
    cid = lax.axis_index("c")
    sid = lax.axis_index("s")
    wid = sid * NC + cid
    row0 = wid * RPW

    rows = (rows0, rows1, rows2, rows3)
    sg = (sg0, sg1, sg2, sg3)
    so = (so0, so1, so2, so3)

    pltpu.sync_copy(enc.at[pl.ds(row0, RPW)], idx_v)

    def fire_gathers(c, b):
        for (off, n) in SPLITS:
            pltpu.async_copy(
                table.at[idx_v.at[c, pl.ds(off, n)]],
                rows[b].at[0, pl.ds(off, n)],
                sg[b],
            )

    def wait_gathers(c, b):
        for (off, n) in SPLITS:
            pltpu.make_async_copy(
                table.at[idx_v.at[c, pl.ds(off, n)]],
                rows[b].at[0, pl.ds(off, n)],
                sg[b],
            ).wait()

    def out_copy(c, b):
        return pltpu.make_async_copy(
            rows[b],
            out.at[pl.ds(row0 + c, 1)],
            so[b],
        )

    for b in range(FIRE_AHEAD):
        fire_gathers(b, b)

    def body(i, carry):
        c0 = i * NBUF
        for b in range(NBUF):
            c = c0 + b
            wait_gathers(c, b)
            out_copy(c, b).start()

            @pl.when(c >= FIRE_AHEAD)
            def _():
                out_copy(c - FIRE_AHEAD, (b + FIRE_AHEAD) % NBUF).wait()

            @pl.when(c + FIRE_AHEAD < NCHUNK)
            def _():
                fire_gathers(c + FIRE_AHEAD, (b + FIRE_AHEAD) % NBUF)

        return carry

    lax.fori_loop(0, NCHUNK // NBUF, body, 0)

    for c in range(NCHUNK - FIRE_AHEAD, NCHUNK):
        out_copy(c, c % NBUF).wait()


def _mask_body(enc_ref, mask_ref):
    mask_ref[...] = (enc_ref[...] != 0).astype(jnp.int32)


TBLK = 32768


def _pad_body(tT_ref, out_ref):
    out_ref[:, :EMB] = jnp.swapaxes(tT_ref[...], 0, 1)


def _make_t128(embeddings):
    grid = (VOCAB + TBLK - 1) // TBLK
    return pl.pallas_call(
        _pad_body,
        grid=(grid,),
        in_specs=[pl.BlockSpec((EMB, TBLK), lambda i: (0, i))],
        out_specs=pl.BlockSpec((TBLK, EMBP), lambda i: (i, 0)),
        out_shape=jax.ShapeDtypeStruct((VOCAB, EMBP), jnp.float32),
    )(embeddings.T)


@jax.jit
def _run(encoded, embeddings):
    t128 = _make_t128(embeddings)
    gather = pl.kernel(
        _gather_kernel,
        out_type=jax.ShapeDtypeStruct((B, L, EMBP), jnp.float32),
        mesh=plsc.VectorSubcoreMesh(core_axis_name="c", subcore_axis_name="s"),
        compiler_params=pltpu.CompilerParams(use_tc_tiling_on_sc=False),
        scratch_types=(
            [pltpu.VMEM((RPW, L), jnp.int32)]
            + [pltpu.VMEM((1, L, EMBP), jnp.float32)
               for _ in range(NBUF)]
            + [pltpu.SemaphoreType.DMA for _ in range(2 * NBUF)]
        ),
    )
    out128 = gather(t128, encoded)
    emb = out128[:, :, :EMB]
    mask = pl.pallas_call(
        _mask_body,
        out_shape=jax.ShapeDtypeStruct((B, L), jnp.int32),
    )(encoded)
    return emb, mask


def kernel(encoded, embeddings):
    return _run(encoded, embeddings)
